# Initial kernel scaffold; baseline (speedup 1.0000x reference)
import jax
import jax.numpy as jnp
from jax.experimental import pallas as pl


def kernel(g, in_feat, W1, b1, W2, b2):
    raise NotImplementedError("write your pallas kernel here")


# speedup vs baseline: 3.2790x; 3.2790x over previous
import functools

import jax
import jax.numpy as jnp
from jax import lax
from jax.experimental import pallas as pl
from jax.experimental.pallas import tpu as pltpu
from jax.experimental.pallas import tpu_sc as plsc

N_NODES = 10000
D = 128
N_EDGES = 320000

NC = 2
NS = 16
NW = NC * NS
CHUNK = 128
K = 79
E_PAD = NW * K * CHUNK
N_ACC = 10112
ROWS_PER_TILE = N_ACC // NS
KPAD = 160
K_STAGE = 80

_MESH = plsc.VectorSubcoreMesh(core_axis_name="c", subcore_axis_name="s")


@functools.partial(
    pl.kernel, mesh=_MESH,
    out_type=jax.ShapeDtypeStruct((NC, N_ACC, D), jnp.float32),
    scratch_types=[
        pltpu.VMEM((K_STAGE, CHUNK), jnp.int32),
        pltpu.VMEM((K_STAGE, CHUNK), jnp.int32),
        pltpu.VMEM((CHUNK, D), jnp.float32),
        pltpu.VMEM_SHARED((N_ACC, D), jnp.float32),
        pltpu.SemaphoreType.DMA,
    ])
def _sc_agg(edges_hbm, x_hbm, agg_out, src_v, dst_v, rows_v, agg_sh, sem):
    cid = lax.axis_index("c")
    sid = lax.axis_index("s")
    wid = cid * NS + sid

    zeros16 = jnp.zeros((16,), jnp.float32)
    for i in range(CHUNK):
        for k in range(D // 16):
            rows_v[i, pl.ds(k * 16, 16)] = zeros16

    pltpu.sync_copy(edges_hbm.at[0, wid, pl.ds(0, K_STAGE)], src_v)
    pltpu.sync_copy(edges_hbm.at[1, wid, pl.ds(0, K_STAGE)], dst_v)

    base = sid * ROWS_PER_TILE
    for k in range(ROWS_PER_TILE // CHUNK):
        pltpu.sync_copy(rows_v, agg_sh.at[pl.ds(base + k * CHUNK, CHUNK)])
    rem = ROWS_PER_TILE % CHUNK
    if rem:
        pltpu.sync_copy(
            rows_v.at[pl.ds(0, rem)],
            agg_sh.at[pl.ds(base + (ROWS_PER_TILE // CHUNK) * CHUNK, rem)])
    plsc.subcore_barrier()

    def edge_body(j, _):
        pltpu.async_copy(x_hbm.at[src_v.at[j]], rows_v, sem).wait()
        pltpu.sync_copy(rows_v, agg_sh.at[dst_v.at[j]], add=True)
        return 0
    lax.fori_loop(0, K, edge_body, 0)
    plsc.subcore_barrier()

    pltpu.sync_copy(agg_sh.at[pl.ds(base, ROWS_PER_TILE)],
                    agg_out.at[cid, pl.ds(base, ROWS_PER_TILE)])


def _tc_combine(parts, deg_parts, x, W, b, relu):
    BN = 400
    grid = (N_NODES // BN,)

    def body(pp, dp, xr, wr, br, orf):
        p = pp[0] + pp[1] + xr[...]
        deg = dp[0, :, :1] + dp[1, :, :1]
        h = p * (1.0 / (deg + 1.0))
        y = lax.dot_general(h, wr[...], (((1,), (1,)), ((), ())),
                            preferred_element_type=jnp.float32)
        y = y + br[...]
        orf[...] = jnp.maximum(y, 0.0) if relu else y

    return pl.pallas_call(
        body,
        grid=grid,
        in_specs=[
            pl.BlockSpec((NC, BN, D), lambda i: (0, i, 0)),
            pl.BlockSpec((NC, BN, D), lambda i: (0, i, 0)),
            pl.BlockSpec((BN, D), lambda i: (i, 0)),
            pl.BlockSpec((D, D), lambda i: (0, 0)),
            pl.BlockSpec((1, D), lambda i: (0, 0)),
        ],
        out_specs=pl.BlockSpec((BN, D), lambda i: (i, 0)),
        out_shape=jax.ShapeDtypeStruct((N_NODES, D), jnp.float32),
    )(parts, deg_parts, x, W, b.reshape(1, D))


def kernel(g, in_feat, W1, b1, W2, b2):
    src = g[0].astype(jnp.int32)
    dst = g[1].astype(jnp.int32)
    pad = E_PAD - N_EDGES
    src_p = jnp.concatenate([src, jnp.zeros((pad,), jnp.int32)]).reshape(NW, K, CHUNK)
    dst_p = jnp.concatenate([dst, jnp.full((pad,), N_NODES, jnp.int32)]).reshape(NW, K, CHUNK)
    kpad = jnp.zeros((NW, KPAD - K, CHUNK), jnp.int32)
    edges = jnp.stack([jnp.concatenate([src_p, kpad], axis=1),
                       jnp.concatenate([dst_p, kpad], axis=1)])

    ones_mat = jnp.ones((N_NODES, D), jnp.float32)
    degp = _sc_agg(edges, ones_mat)
    agg1 = _sc_agg(edges, in_feat)
    h1 = _tc_combine(agg1, degp, in_feat, W1, b1, relu=True)
    agg2 = _sc_agg(edges, h1)
    return _tc_combine(agg2, degp, h1, W2, b2, relu=False)
